# baseline (speedup 1.0000x reference)
import functools

import jax
import jax.numpy as jnp
from jax import lax
from jax.experimental import pallas as pl
from jax.experimental.pallas import tpu as pltpu
from jax.experimental.pallas import tpu_sc as plsc

N = 4096
D_PAD = 384
S_COLS = 1024
NC, NS = 2, 16
NW = NC * NS
ROWS_W = N // NW
GRP = 8
NGRP = ROWS_W // GRP


def _matmul_body(u_ref, v_ref, s_ref):
    acc = lax.dot_general(
        u_ref[...], v_ref[...], (((1,), (1,)), ((), ())),
        preferred_element_type=jnp.float32)
    s_ref[...] = jax.nn.sigmoid(acc)


def _sc_body(s_hbm, a_hbm, b_hbm, out_hbm,
             idx_v, b_v, st0, st1, ob0, ob1, gsem, ssem):
    wid = lax.axis_index("s") * NC + lax.axis_index("c")
    base = wid * ROWS_W
    pltpu.sync_copy(a_hbm.at[pl.ds(base, ROWS_W)], idx_v)
    pltpu.sync_copy(b_hbm, b_v)

    stages = (st0, st1)
    obufs = (ob0, ob1)
    row_ids = [jnp.full((16,), r, jnp.int32) for r in range(GRP)]

    def fire(g):
        return pltpu.async_copy(
            s_hbm.at[idx_v.at[pl.ds(g * GRP, GRP)]], stages[g % 2], gsem)

    def put(g):
        return pltpu.async_copy(
            obufs[g % 2], out_hbm.at[pl.ds(base + g * GRP, GRP)], ssem)

    def assemble(g):
        st = stages[g % 2]
        ob = obufs[g % 2]

        def body(t, carry):
            off = pl.multiple_of(t * 16, 16)
            cols = b_v[pl.ds(off, 16)]
            for r in range(GRP):
                ob[r, pl.ds(off, 16)] = plsc.load_gather(
                    st, [row_ids[r], cols])
            return carry

        lax.fori_loop(0, N // 16, body, 0)

    g_cp = {}
    s_cp = {}
    g_cp[0] = fire(0)
    for g in range(NGRP):
        if g + 1 < NGRP:
            g_cp[g + 1] = fire(g + 1)
        g_cp[g].wait()
        if g >= 2:
            s_cp[g - 2].wait()
        assemble(g)
        s_cp[g] = put(g)
    for g in range(max(0, NGRP - 2), NGRP):
        s_cp[g].wait()


def kernel(X, U, V):
    a = X[:, 0]
    b = X[:, 1]
    vocab, d = U.shape
    u_pad = jnp.pad(U, ((0, 0), (0, D_PAD - d)))
    v_pad = jnp.pad(V, ((0, S_COLS - vocab), (0, D_PAD - d)))

    s = pl.pallas_call(
        _matmul_body,
        grid=(1,),
        in_specs=[
            pl.BlockSpec((vocab, D_PAD), lambda j: (0, 0)),
            pl.BlockSpec((S_COLS, D_PAD), lambda j: (0, 0)),
        ],
        out_specs=pl.BlockSpec((vocab, S_COLS), lambda j: (0, 0)),
        out_shape=jax.ShapeDtypeStruct((vocab, S_COLS), jnp.float32),
    )(u_pad, v_pad)

    mesh = plsc.VectorSubcoreMesh(core_axis_name="c", subcore_axis_name="s")
    out = pl.kernel(
        _sc_body,
        out_type=jax.ShapeDtypeStruct((N, N), jnp.float32),
        mesh=mesh,
        compiler_params=pltpu.CompilerParams(needs_layout_passes=False),
        scratch_types=[
            pltpu.VMEM((ROWS_W,), jnp.int32),
            pltpu.VMEM((N,), jnp.int32),
            pltpu.VMEM((GRP, S_COLS), jnp.float32),
            pltpu.VMEM((GRP, S_COLS), jnp.float32),
            pltpu.VMEM((GRP, N), jnp.float32),
            pltpu.VMEM((GRP, N), jnp.float32),
            pltpu.SemaphoreType.DMA,
            pltpu.SemaphoreType.DMA,
        ],
    )(s, a, b)
    return out


# speedup vs baseline: 2.9386x; 2.9386x over previous
import functools

import jax
import jax.numpy as jnp
from jax import lax
from jax.experimental import pallas as pl
from jax.experimental.pallas import tpu as pltpu
from jax.experimental.pallas import tpu_sc as plsc

N = 4096
D_PAD = 384
BLK = 512
NC, NS = 2, 16
NW = NC * NS
ROWS_W = 2 * N // NW


def _sc_gather_body(table, idx, out, idx_v, buf_v, gsem):
    wid = lax.axis_index("s") * NC + lax.axis_index("c")
    base = wid * ROWS_W
    pltpu.sync_copy(idx.at[pl.ds(base, ROWS_W)], idx_v)
    pltpu.async_copy(table.at[idx_v], buf_v, gsem).wait()
    pltpu.sync_copy(buf_v, out.at[pl.ds(base, ROWS_W)])


def _matmul_body(ua_ref, vb_ref, o_ref, ua_bf):
    @pl.when(pl.program_id(0) == 0)
    def _():
        ua_bf[...] = ua_ref[...].astype(jnp.bfloat16)

    vb = vb_ref[...].astype(jnp.bfloat16)
    acc = lax.dot_general(
        ua_bf[...], vb, (((1,), (1,)), ((), ())),
        preferred_element_type=jnp.float32)
    o_ref[...] = jax.nn.sigmoid(acc)


def kernel(X, U, V):
    vocab, d = U.shape
    idx = jnp.concatenate([X[:, 0], X[:, 1] + vocab])
    table = jnp.pad(
        jnp.concatenate([U, V], axis=0), ((0, 0), (0, D_PAD - d)))

    mesh = plsc.VectorSubcoreMesh(core_axis_name="c", subcore_axis_name="s")
    g = pl.kernel(
        _sc_gather_body,
        out_type=jax.ShapeDtypeStruct((2 * N, D_PAD), jnp.float32),
        mesh=mesh,
        compiler_params=pltpu.CompilerParams(needs_layout_passes=False),
        scratch_types=[
            pltpu.VMEM((ROWS_W,), jnp.int32),
            pltpu.VMEM((ROWS_W, D_PAD), jnp.float32),
            pltpu.SemaphoreType.DMA,
        ],
    )(table, idx)

    out = pl.pallas_call(
        _matmul_body,
        grid=(N // BLK,),
        in_specs=[
            pl.BlockSpec((N, D_PAD), lambda j: (0, 0)),
            pl.BlockSpec((BLK, D_PAD), lambda j: (N // BLK + j, 0)),
        ],
        out_specs=pl.BlockSpec((N, BLK), lambda j: (0, j)),
        out_shape=jax.ShapeDtypeStruct((N, N), jnp.float32),
        scratch_shapes=[pltpu.VMEM((N, D_PAD), jnp.bfloat16)],
    )(g, g)
    return out
